# baseline (speedup 1.0000x reference)
import functools

import jax
import jax.numpy as jnp
from jax import lax
from jax.experimental import pallas as pl
from jax.experimental.pallas import tpu as pltpu, tpu_sc as plsc

F = 16
LANES = 16
CHUNK = 8192

_PI_HI = 3.14159274101257324
_INV_PI = 0.318309886183790672
_RND = 12582912.0
_S0 = 9.99999997e-01
_S1 = -1.66666600e-01
_S2 = 8.33309759e-03
_S3 = -1.98124878e-04
_S4 = 2.61290778e-06


def _sin(t):
    q = t * _INV_PI
    m = q + _RND
    nf = m - _RND
    y = t - nf * _PI_HI
    y2 = y * y
    s = y * (_S0 + y2 * (_S1 + y2 * (_S2 + y2 * (_S3 + y2 * _S4))))
    flip = plsc.bitcast(m, jnp.int32) << 31
    return plsc.bitcast(plsc.bitcast(s, jnp.int32) ^ flip, jnp.float32)


def _make_sc_kernel(n_points):
    info = plsc.get_sparse_core_info()
    nc, ns = info.num_cores, info.num_subcores
    nw = nc * ns
    per_w = n_points // nw
    n_chunks = per_w // CHUNK
    mesh = plsc.VectorSubcoreMesh(core_axis_name="c", subcore_axis_name="s")
    col = jax.ShapeDtypeStruct((n_points,), jnp.float32)
    fbuf = pltpu.VMEM((CHUNK,), jnp.float32)
    ibuf = pltpu.VMEM((CHUNK,), jnp.int32)

    @functools.partial(
        pl.kernel,
        mesh=mesh,
        out_type=(col, col, col),
        compiler_params=pltpu.CompilerParams(needs_layout_passes=False),
        scratch_types=[
            pltpu.VMEM((10 * F,), jnp.float32),
            fbuf, fbuf, ibuf, fbuf, fbuf, fbuf, fbuf,
            fbuf, fbuf, ibuf, fbuf, fbuf, fbuf, fbuf,
            pltpu.SemaphoreType.DMA,
            pltpu.SemaphoreType.DMA,
            pltpu.SemaphoreType.DMA,
            pltpu.SemaphoreType.DMA,
        ],
    )
    def sc_kernel(x_hbm, y_hbm, c_hbm, ch_hbm, par_hbm,
                  ox_hbm, oy_hbm, oc_hbm,
                  par_v,
                  x0, y0, ch0, c0, ox0, oy0, oc0,
                  x1, y1, ch1, c1, ox1, oy1, oc1,
                  isem0, isem1, osem0, osem1):
        wid = lax.axis_index("s") * nc + lax.axis_index("c")
        pltpu.sync_copy(par_hbm, par_v)
        bufs = ((x0, y0, ch0, c0, ox0, oy0, oc0, isem0, osem0),
                (x1, y1, ch1, c1, ox1, oy1, oc1, isem1, osem1))

        def start_in(chunk):
            x_v, y_v, ch_v, c_v, _, _, _, isem, _ = bufs[chunk % 2]
            base = wid * per_w + chunk * CHUNK
            sl = pl.ds(base, CHUNK)
            return (pltpu.async_copy(x_hbm.at[sl], x_v, isem),
                    pltpu.async_copy(y_hbm.at[sl], y_v, isem),
                    pltpu.async_copy(ch_hbm.at[sl], ch_v, isem),
                    pltpu.async_copy(c_hbm.at[sl], c_v, isem))

        def start_out(chunk):
            _, _, _, _, ox_v, oy_v, oc_v, _, osem = bufs[chunk % 2]
            base = wid * per_w + chunk * CHUNK
            sl = pl.ds(base, CHUNK)
            return (pltpu.async_copy(ox_v, ox_hbm.at[sl], osem),
                    pltpu.async_copy(oy_v, oy_hbm.at[sl], osem),
                    pltpu.async_copy(oc_v, oc_hbm.at[sl], osem))

        def compute(chunk):
            x_v, y_v, ch_v, c_v, ox_v, oy_v, oc_v, _, _ = bufs[chunk % 2]
            tabs = [par_v[pl.ds(k * F, F)] for k in range(10)]

            def take(tbl, ch):
                return tbl.at[ch].get(mode="promise_in_bounds")

            @plsc.parallel_loop(0, CHUNK, step=LANES, unroll=2)
            def _body(k0):
                x = x_v[pl.ds(k0, LANES)]
                y = y_v[pl.ds(k0, LANES)]
                ch = ch_v[pl.ds(k0, LANES)]
                a00 = take(tabs[0], ch)
                a01 = take(tabs[1], ch)
                a10 = take(tabs[2], ch)
                a11 = take(tabs[3], ch)
                b0 = take(tabs[4], ch)
                b1 = take(tabs[5], ch)
                w0 = take(tabs[6], ch)
                w1 = take(tabs[7], ch)
                w2 = take(tabs[8], ch)
                fc = take(tabs[9], ch)
                tx = a00 * x + a01 * y + b0
                ty = a10 * x + a11 * y + b1
                r2 = tx * tx + ty * ty + 1e-6
                g = w0 + w2 * (1.0 / r2)
                ox_v[pl.ds(k0, LANES)] = tx * g + w1 * _sin(tx)
                oy_v[pl.ds(k0, LANES)] = ty * g + w1 * _sin(ty)
                oc_v[pl.ds(k0, LANES)] = (c_v[pl.ds(k0, LANES)] + fc) * 0.5

        in_h = {0: start_in(0)}
        out_h = {}
        for k in range(n_chunks):
            if k + 1 < n_chunks:
                in_h[k + 1] = start_in(k + 1)
            for h in in_h.pop(k):
                h.wait()
            if k >= 2:
                for h in out_h.pop(k - 2):
                    h.wait()
            compute(k)
            out_h[k] = start_out(k)
        for hs in out_h.values():
            for h in hs:
                h.wait()

    return sc_kernel


def kernel(points, A, b, vweights, colors, choices):
    n = points.shape[0]
    params = jnp.concatenate([
        A.reshape(F, 4).T.reshape(-1),
        b.T.reshape(-1),
        vweights.T.reshape(-1),
        colors,
    ]).astype(jnp.float32)
    ch = choices.astype(jnp.int32)
    ox, oy, oc = _make_sc_kernel(n)(
        points[:, 0], points[:, 1], points[:, 2], ch, params)
    return jnp.stack([ox, oy, oc], axis=1)


# speedup vs baseline: 1.0182x; 1.0182x over previous
import functools

import jax
import jax.numpy as jnp
from jax import lax
from jax.experimental import pallas as pl
from jax.experimental.pallas import tpu as pltpu, tpu_sc as plsc

F = 16
LANES = 16
CHUNK = 8192

_PI_HI = 3.14159274101257324
_INV_PI = 0.318309886183790672
_RND = 12582912.0
_S0 = 9.99999997e-01
_S1 = -1.66666600e-01
_S2 = 8.33309759e-03
_S3 = -1.98124878e-04
_S4 = 2.61290778e-06


def _sin(t):
    q = t * _INV_PI
    m = q + _RND
    nf = m - _RND
    y = t - nf * _PI_HI
    y2 = y * y
    s = y * (_S0 + y2 * (_S1 + y2 * (_S2 + y2 * (_S3 + y2 * _S4))))
    flip = plsc.bitcast(m, jnp.int32) << 31
    return plsc.bitcast(plsc.bitcast(s, jnp.int32) ^ flip, jnp.float32)


def _make_sc_kernel(n_points):
    info = plsc.get_sparse_core_info()
    nc, ns = info.num_cores, info.num_subcores
    nw = nc * ns
    per_w = n_points // nw
    n_chunks = per_w // CHUNK
    mesh = plsc.VectorSubcoreMesh(core_axis_name="c", subcore_axis_name="s")
    col = jax.ShapeDtypeStruct((n_points,), jnp.float32)
    fbuf = pltpu.VMEM((CHUNK,), jnp.float32)
    ibuf = pltpu.VMEM((CHUNK,), jnp.int32)

    @functools.partial(
        pl.kernel,
        mesh=mesh,
        out_type=(col, col, col),
        compiler_params=pltpu.CompilerParams(needs_layout_passes=False),
        scratch_types=[
            pltpu.VMEM((10 * F,), jnp.float32),
            fbuf, fbuf, ibuf, fbuf, fbuf, fbuf, fbuf,
            fbuf, fbuf, ibuf, fbuf, fbuf, fbuf, fbuf,
            pltpu.SemaphoreType.DMA,
            pltpu.SemaphoreType.DMA,
            pltpu.SemaphoreType.DMA,
            pltpu.SemaphoreType.DMA,
        ],
    )
    def sc_kernel(x_hbm, y_hbm, c_hbm, ch_hbm, par_hbm,
                  ox_hbm, oy_hbm, oc_hbm,
                  par_v,
                  x0, y0, ch0, c0, ox0, oy0, oc0,
                  x1, y1, ch1, c1, ox1, oy1, oc1,
                  isem0, isem1, osem0, osem1):
        wid = lax.axis_index("s") * nc + lax.axis_index("c")
        pltpu.sync_copy(par_hbm, par_v)
        bufs = ((x0, y0, ch0, c0, ox0, oy0, oc0, isem0, osem0),
                (x1, y1, ch1, c1, ox1, oy1, oc1, isem1, osem1))

        def start_in(chunk):
            x_v, y_v, ch_v, c_v, _, _, _, isem, _ = bufs[chunk % 2]
            base = wid * per_w + chunk * CHUNK
            sl = pl.ds(base, CHUNK)
            return (pltpu.async_copy(x_hbm.at[sl], x_v, isem),
                    pltpu.async_copy(y_hbm.at[sl], y_v, isem),
                    pltpu.async_copy(ch_hbm.at[sl], ch_v, isem),
                    pltpu.async_copy(c_hbm.at[sl], c_v, isem))

        def start_out(chunk):
            _, _, _, _, ox_v, oy_v, oc_v, _, osem = bufs[chunk % 2]
            base = wid * per_w + chunk * CHUNK
            sl = pl.ds(base, CHUNK)
            return (pltpu.async_copy(ox_v, ox_hbm.at[sl], osem),
                    pltpu.async_copy(oy_v, oy_hbm.at[sl], osem),
                    pltpu.async_copy(oc_v, oc_hbm.at[sl], osem))

        def compute(chunk):
            x_v, y_v, ch_v, c_v, ox_v, oy_v, oc_v, _, _ = bufs[chunk % 2]
            @plsc.parallel_loop(0, CHUNK, step=LANES, unroll=2)
            def _body(k0):
                x = x_v[pl.ds(k0, LANES)]
                y = y_v[pl.ds(k0, LANES)]
                ch = ch_v[pl.ds(k0, LANES)]
                a00 = plsc.load_gather(par_v, [ch])
                a01 = plsc.load_gather(par_v, [ch + F])
                a10 = plsc.load_gather(par_v, [ch + 2 * F])
                a11 = plsc.load_gather(par_v, [ch + 3 * F])
                b0 = plsc.load_gather(par_v, [ch + 4 * F])
                b1 = plsc.load_gather(par_v, [ch + 5 * F])
                w0 = plsc.load_gather(par_v, [ch + 6 * F])
                w1 = plsc.load_gather(par_v, [ch + 7 * F])
                w2 = plsc.load_gather(par_v, [ch + 8 * F])
                fc = plsc.load_gather(par_v, [ch + 9 * F])
                tx = a00 * x + a01 * y + b0
                ty = a10 * x + a11 * y + b1
                r2 = tx * tx + ty * ty + 1e-6
                g = w0 + w2 * (1.0 / r2)
                ox_v[pl.ds(k0, LANES)] = tx * g + w1 * _sin(tx)
                oy_v[pl.ds(k0, LANES)] = ty * g + w1 * _sin(ty)
                oc_v[pl.ds(k0, LANES)] = (c_v[pl.ds(k0, LANES)] + fc) * 0.5

        in_h = {0: start_in(0)}
        out_h = {}
        for k in range(n_chunks):
            if k + 1 < n_chunks:
                in_h[k + 1] = start_in(k + 1)
            for h in in_h.pop(k):
                h.wait()
            if k >= 2:
                for h in out_h.pop(k - 2):
                    h.wait()
            compute(k)
            out_h[k] = start_out(k)
        for hs in out_h.values():
            for h in hs:
                h.wait()

    return sc_kernel


def kernel(points, A, b, vweights, colors, choices):
    n = points.shape[0]
    params = jnp.concatenate([
        A.reshape(F, 4).T.reshape(-1),
        b.T.reshape(-1),
        vweights.T.reshape(-1),
        colors,
    ]).astype(jnp.float32)
    ch = choices.astype(jnp.int32)
    ox, oy, oc = _make_sc_kernel(n)(
        points[:, 0], points[:, 1], points[:, 2], ch, params)
    return jnp.stack([ox, oy, oc], axis=1)


# speedup vs baseline: 1.0379x; 1.0194x over previous
import functools

import jax
import jax.numpy as jnp
from jax import lax
from jax.experimental import pallas as pl
from jax.experimental.pallas import tpu as pltpu, tpu_sc as plsc

F = 16
LANES = 16
CHUNK = 4096

_PI_HI = 3.14159274101257324
_INV_PI = 0.318309886183790672
_RND = 12582912.0
_S0 = 9.99999470e-01
_S1 = -1.66658913e-01
_S2 = 8.31596486e-03
_S3 = -1.86089757e-04


def _sin(t):
    q = t * _INV_PI
    m = q + _RND
    nf = m - _RND
    y = t - nf * _PI_HI
    y2 = y * y
    s = y * (_S0 + y2 * (_S1 + y2 * (_S2 + y2 * _S3)))
    flip = plsc.bitcast(m, jnp.int32) << 31
    return plsc.bitcast(plsc.bitcast(s, jnp.int32) ^ flip, jnp.float32)


def _make_sc_kernel(n_points):
    info = plsc.get_sparse_core_info()
    nc, ns = info.num_cores, info.num_subcores
    nw = nc * ns
    per_w = n_points // nw
    n_chunks = per_w // CHUNK
    mesh = plsc.VectorSubcoreMesh(core_axis_name="c", subcore_axis_name="s")
    col = jax.ShapeDtypeStruct((n_points,), jnp.float32)
    fbuf = pltpu.VMEM((CHUNK,), jnp.float32)
    ibuf = pltpu.VMEM((CHUNK,), jnp.int32)

    @functools.partial(
        pl.kernel,
        mesh=mesh,
        out_type=(col, col, col),
        compiler_params=pltpu.CompilerParams(needs_layout_passes=False),
        scratch_types=[
            pltpu.VMEM((10 * F,), jnp.float32),
            fbuf, fbuf, ibuf, fbuf, fbuf, fbuf, fbuf,
            fbuf, fbuf, ibuf, fbuf, fbuf, fbuf, fbuf,
            pltpu.SemaphoreType.DMA,
            pltpu.SemaphoreType.DMA,
            pltpu.SemaphoreType.DMA,
            pltpu.SemaphoreType.DMA,
        ],
    )
    def sc_kernel(x_hbm, y_hbm, c_hbm, ch_hbm, par_hbm,
                  ox_hbm, oy_hbm, oc_hbm,
                  par_v,
                  x0, y0, ch0, c0, ox0, oy0, oc0,
                  x1, y1, ch1, c1, ox1, oy1, oc1,
                  isem0, isem1, osem0, osem1):
        wid = lax.axis_index("s") * nc + lax.axis_index("c")
        pltpu.sync_copy(par_hbm, par_v)
        bufs = ((x0, y0, ch0, c0, ox0, oy0, oc0, isem0, osem0),
                (x1, y1, ch1, c1, ox1, oy1, oc1, isem1, osem1))

        def start_in(chunk):
            x_v, y_v, ch_v, c_v, _, _, _, isem, _ = bufs[chunk % 2]
            base = wid * per_w + chunk * CHUNK
            sl = pl.ds(base, CHUNK)
            return (pltpu.async_copy(x_hbm.at[sl], x_v, isem),
                    pltpu.async_copy(y_hbm.at[sl], y_v, isem),
                    pltpu.async_copy(ch_hbm.at[sl], ch_v, isem),
                    pltpu.async_copy(c_hbm.at[sl], c_v, isem))

        def start_out(chunk):
            _, _, _, _, ox_v, oy_v, oc_v, _, osem = bufs[chunk % 2]
            base = wid * per_w + chunk * CHUNK
            sl = pl.ds(base, CHUNK)
            return (pltpu.async_copy(ox_v, ox_hbm.at[sl], osem),
                    pltpu.async_copy(oy_v, oy_hbm.at[sl], osem),
                    pltpu.async_copy(oc_v, oc_hbm.at[sl], osem))

        def compute(chunk):
            x_v, y_v, ch_v, c_v, ox_v, oy_v, oc_v, _, _ = bufs[chunk % 2]
            @plsc.parallel_loop(0, CHUNK, step=LANES, unroll=2)
            def _body(k0):
                x = x_v[pl.ds(k0, LANES)]
                y = y_v[pl.ds(k0, LANES)]
                ch = ch_v[pl.ds(k0, LANES)]
                a00 = plsc.load_gather(par_v, [ch])
                a01 = plsc.load_gather(par_v, [ch + F])
                a10 = plsc.load_gather(par_v, [ch + 2 * F])
                a11 = plsc.load_gather(par_v, [ch + 3 * F])
                b0 = plsc.load_gather(par_v, [ch + 4 * F])
                b1 = plsc.load_gather(par_v, [ch + 5 * F])
                w0 = plsc.load_gather(par_v, [ch + 6 * F])
                w1 = plsc.load_gather(par_v, [ch + 7 * F])
                w2 = plsc.load_gather(par_v, [ch + 8 * F])
                fc = plsc.load_gather(par_v, [ch + 9 * F])
                tx = a00 * x + a01 * y + b0
                ty = a10 * x + a11 * y + b1
                r2 = tx * tx + ty * ty + 1e-6
                g = w0 + w2 * (1.0 / r2)
                ox_v[pl.ds(k0, LANES)] = tx * g + w1 * _sin(tx)
                oy_v[pl.ds(k0, LANES)] = ty * g + w1 * _sin(ty)
                oc_v[pl.ds(k0, LANES)] = (c_v[pl.ds(k0, LANES)] + fc) * 0.5

        in_h = {0: start_in(0)}
        out_h = {}
        for k in range(n_chunks):
            if k + 1 < n_chunks:
                in_h[k + 1] = start_in(k + 1)
            for h in in_h.pop(k):
                h.wait()
            if k >= 2:
                for h in out_h.pop(k - 2):
                    h.wait()
            compute(k)
            out_h[k] = start_out(k)
        for hs in out_h.values():
            for h in hs:
                h.wait()

    return sc_kernel


def kernel(points, A, b, vweights, colors, choices):
    n = points.shape[0]
    params = jnp.concatenate([
        A.reshape(F, 4).T.reshape(-1),
        b.T.reshape(-1),
        vweights.T.reshape(-1),
        colors,
    ]).astype(jnp.float32)
    ch = choices.astype(jnp.int32)
    ox, oy, oc = _make_sc_kernel(n)(
        points[:, 0], points[:, 1], points[:, 2], ch, params)
    return jnp.stack([ox, oy, oc], axis=1)
